# Initial kernel scaffold; baseline (speedup 1.0000x reference)
import jax
import jax.numpy as jnp
from jax.experimental import pallas as pl


def kernel(x, pos_emb):
    raise NotImplementedError("write your pallas kernel here")


# speedup vs baseline: 13.8040x; 13.8040x over previous
import jax
import jax.numpy as jnp
from jax.experimental import pallas as pl


def _bcast_body(tab_ref, out_ref):
    out_ref[...] = jnp.broadcast_to(tab_ref[...], out_ref.shape)


def kernel(x, pos_emb):
    B = x.shape[0]
    S, E = pos_emb.shape
    D = S * E
    tab = pos_emb.reshape(1, D)
    BB = 128
    out = pl.pallas_call(
        _bcast_body,
        grid=(B // BB,),
        in_specs=[pl.BlockSpec((1, D), lambda i: (0, 0))],
        out_specs=pl.BlockSpec((BB, D), lambda i: (i, 0)),
        out_shape=jax.ShapeDtypeStruct((B, D), jnp.float32),
    )(tab)
    return out.reshape(B, S, E)
